# baseline (speedup 1.0000x reference)
import functools

import jax
import jax.numpy as jnp
from jax import lax
from jax.experimental import pallas as pl
from jax.experimental.pallas import tpu as pltpu
from jax.experimental.pallas import tpu_sc as plsc

_N_NODES = 10000
_N_EDGES = 320000
_D = 128
_N_GRAPHS = 128

_NC = 2
_NS = 16
_EB = 50
_NBUF = 5
_QG = 40
_EROWS = _N_EDGES // _EB
_EROWS_TILE = _EROWS // (_NC * _NS)
_N_PAD = 10240
_NROWS_TILE = _N_PAD // _NS

_sc_mesh = plsc.VectorSubcoreMesh(core_axis_name="c", subcore_axis_name="s")


@functools.partial(
    pl.kernel,
    mesh=_sc_mesh,
    out_type=jax.ShapeDtypeStruct((_NC, _N_PAD, _D), jnp.float32),
    scratch_types=[
        pltpu.VMEM((_QG, _EB), jnp.int32),
        pltpu.VMEM((_QG, _EB), jnp.int32),
        pltpu.VMEM((_NBUF, _EB, _D), jnp.float32),
        pltpu.VMEM_SHARED((_N_PAD, _D), jnp.float32),
        pltpu.SemaphoreType.DMA,
        pltpu.SemaphoreType.DMA,
        pltpu.SemaphoreType.DMA,
        pltpu.SemaphoreType.DMA,
        pltpu.SemaphoreType.DMA,
    ],
)
def _sc_aggregate(x_hbm, src_hbm, dst_hbm, zeros_hbm, out_hbm,
                  idx_s_v, idx_d_v, rows_v, acc_sh, s0, s1, s2, s3, s4):
    c = lax.axis_index("c")
    s = lax.axis_index("s")
    sems = (s0, s1, s2, s3, s4)
    base = (c * _NS + s) * _EROWS_TILE

    nbase = s * _NROWS_TILE
    pltpu.sync_copy(zeros_hbm, acc_sh.at[pl.ds(nbase, _NROWS_TILE)])
    plsc.subcore_barrier()

    def gath(g, b):
        pltpu.async_copy(x_hbm.at[idx_s_v.at[g]], rows_v.at[b], sems[b])

    def wait(b):
        pltpu.make_async_copy(x_hbm.at[idx_s_v.at[b]], rows_v.at[b],
                              sems[b]).wait()

    def scat(g, b):
        pltpu.sync_copy(rows_v.at[b], acc_sh.at[idx_d_v.at[g]], add=True)

    for q in range(_EROWS_TILE // _QG):
        pltpu.sync_copy(src_hbm.at[pl.ds(base + q * _QG, _QG)], idx_s_v)
        pltpu.sync_copy(dst_hbm.at[pl.ds(base + q * _QG, _QG)], idx_d_v)

        for b in range(_NBUF):
            gath(b, b)

        def body(i, carry):
            for b in range(_NBUF):
                g = i * _NBUF + b
                wait(b)
                scat(g, b)
                gath(g + _NBUF, b)
            return carry

        lax.fori_loop(0, _QG // _NBUF - 2, body, 0)
        g0 = _QG - 2 * _NBUF
        for b in range(_NBUF):
            wait(b)
            scat(g0 + b, b)
            gath(g0 + _NBUF + b, b)
        for b in range(_NBUF):
            wait(b)
            scat(g0 + _NBUF + b, b)
    plsc.subcore_barrier()

    pltpu.sync_copy(acc_sh.at[pl.ds(nbase, _NROWS_TILE)],
                    out_hbm.at[c, pl.ds(nbase, _NROWS_TILE)])


_BLK = 2000


def _mlp1_body(x_ref, agg_ref, w_ref, b_ref, o_ref):
    h = x_ref[...] + agg_ref[0] + agg_ref[1]
    y = jnp.dot(h, w_ref[...], preferred_element_type=jnp.float32) + b_ref[...]
    o_ref[...] = jnp.where(y >= 0, y, 0.01 * y)


def _mlp2_pool_body(h1_ref, agg_ref, batch_ref, w2_ref, b2_ref, wf_ref,
                    bf_ref, o_ref, sums, cnts):
    i = pl.program_id(0)
    h = h1_ref[...] + agg_ref[0] + agg_ref[1]
    y = jnp.dot(h, w2_ref[...], preferred_element_type=jnp.float32) + b2_ref[...]
    h2 = jnp.where(y >= 0, y, 0.01 * y)

    bids = batch_ref[0]
    gid = lax.broadcasted_iota(jnp.int32, (_N_GRAPHS, _BLK), 0)
    st = (bids == gid).astype(jnp.float32)
    ps = lax.dot_general(st, h2, (((1,), (0,)), ((), ())),
                         preferred_element_type=jnp.float32)
    pc = lax.dot_general(st, jnp.ones_like(h2), (((1,), (0,)), ((), ())),
                         preferred_element_type=jnp.float32)

    @pl.when(i == 0)
    def _init():
        sums[...] = jnp.zeros_like(sums)
        cnts[...] = jnp.zeros_like(cnts)

    sums[...] += ps
    cnts[...] += pc

    @pl.when(i == pl.num_programs(0) - 1)
    def _fin():
        mean = sums[...] / jnp.maximum(cnts[...], 1.0)
        nrm = jnp.sqrt(jnp.sum(mean * mean, axis=1, keepdims=True))
        hg = mean / jnp.maximum(nrm, 1e-12)
        o_ref[...] = (jnp.dot(hg, wf_ref[...],
                              preferred_element_type=jnp.float32) + bf_ref[...])


def kernel(x, edge_index, batch, W1, b1, W2, b2, Wf, bf):
    src = edge_index[0].astype(jnp.int32).reshape(_EROWS, _EB)
    dst = edge_index[1].astype(jnp.int32).reshape(_EROWS, _EB)
    zeros = jnp.zeros((_NROWS_TILE, _D), jnp.float32)
    b1r = b1.reshape(1, _D)
    b2r = b2.reshape(1, _D)
    bfr = bf.reshape(1, _D)
    batch3 = batch.astype(jnp.int32).reshape(_N_NODES // _BLK, 1, _BLK)

    agg1 = _sc_aggregate(x, src, dst, zeros)

    grid = _N_NODES // _BLK
    h1 = pl.pallas_call(
        _mlp1_body,
        grid=(grid,),
        in_specs=[
            pl.BlockSpec((_BLK, _D), lambda i: (i, 0)),
            pl.BlockSpec((_NC, _BLK, _D), lambda i: (0, i, 0)),
            pl.BlockSpec((_D, _D), lambda i: (0, 0)),
            pl.BlockSpec((1, _D), lambda i: (0, 0)),
        ],
        out_specs=pl.BlockSpec((_BLK, _D), lambda i: (i, 0)),
        out_shape=jax.ShapeDtypeStruct((_N_NODES, _D), jnp.float32),
    )(x, agg1, W1, b1r)

    agg2 = _sc_aggregate(h1, src, dst, zeros)

    out = pl.pallas_call(
        _mlp2_pool_body,
        grid=(grid,),
        in_specs=[
            pl.BlockSpec((_BLK, _D), lambda i: (i, 0)),
            pl.BlockSpec((_NC, _BLK, _D), lambda i: (0, i, 0)),
            pl.BlockSpec((1, 1, _BLK), lambda i: (i, 0, 0)),
            pl.BlockSpec((_D, _D), lambda i: (0, 0)),
            pl.BlockSpec((1, _D), lambda i: (0, 0)),
            pl.BlockSpec((_D, _D), lambda i: (0, 0)),
            pl.BlockSpec((1, _D), lambda i: (0, 0)),
        ],
        out_specs=pl.BlockSpec((_N_GRAPHS, _D), lambda i: (0, 0)),
        out_shape=jax.ShapeDtypeStruct((_N_GRAPHS, _D), jnp.float32),
        scratch_shapes=[
            pltpu.VMEM((_N_GRAPHS, _D), jnp.float32),
            pltpu.VMEM((_N_GRAPHS, _D), jnp.float32),
        ],
    )(h1, agg2, batch3, W2, b2r, Wf, bfr)
    return out
